# baseline (speedup 1.0000x reference)
import functools

import jax
import jax.numpy as jnp
from jax import lax
from jax.experimental import pallas as pl
from jax.experimental.pallas import tpu as pltpu
from jax.experimental.pallas import tpu_sc as plsc

N = 10000
E = 320000
D = 128
DE = 16
NCLS = 2

NC = 2
NS = 16
NW = NC * NS
CH = 128
NCHUNK = 79
EPW = NCHUNK * CH
E_PAD = NW * EPW
NCKA = 53
NCKB = 105
R = 10112
RPT = R // NS

_mesh = plsc.VectorSubcoreMesh(core_axis_name="c", subcore_axis_name="s")


@functools.partial(
    pl.kernel, mesh=_mesh,
    out_type=jax.ShapeDtypeStruct((NC, R, D), jnp.float32),
    scratch_types=[
        pltpu.VMEM((2, 2, CH), jnp.int32),
        pltpu.VMEM((2, CH, D), jnp.float32),
        pltpu.VMEM_SHARED((R, D), jnp.float32),
        pltpu.SemaphoreType.DMA,
        pltpu.SemaphoreType.DMA,
    ],
)
def _sc_stats(idx_hbm, aug_hbm, zh_hbm, out_st, idxv, rowsv, acc, sem0, sem1):
    c = lax.axis_index("c")
    s = lax.axis_index("s")
    w = c * NS + s
    r0 = s * RPT
    pltpu.sync_copy(zh_hbm.at[pl.ds(r0, RPT)], acc.at[pl.ds(r0, RPT)])
    plsc.subcore_barrier()

    ebase = w * EPW
    sems = (sem0, sem1)
    pltpu.sync_copy(idx_hbm.at[w, 0], idxv.at[0])
    pltpu.async_copy(aug_hbm.at[pl.ds(ebase, CH)], rowsv.at[0], sem0)

    def body(k, carry):
        for b in range(2):
            nb = 1 - b
            j = k * 2 + b
            pltpu.async_copy(aug_hbm.at[pl.ds(ebase + (j + 1) * CH, CH)],
                             rowsv.at[nb], sems[nb])
            pltpu.sync_copy(idx_hbm.at[w, j + 1], idxv.at[nb])
            pltpu.make_async_copy(
                aug_hbm.at[pl.ds(ebase, CH)], rowsv.at[b], sems[b]).wait()
            pltpu.sync_copy(rowsv.at[b], acc.at[idxv.at[b, 1]], add=True)
        return carry

    lax.fori_loop(0, (NCHUNK - 1) // 2, body, 0)
    pltpu.make_async_copy(aug_hbm.at[pl.ds(ebase, CH)], rowsv.at[0],
                          sem0).wait()
    pltpu.sync_copy(rowsv.at[0], acc.at[idxv.at[0, 1]], add=True)
    plsc.subcore_barrier()
    pltpu.sync_copy(acc.at[pl.ds(r0, RPT)], out_st.at[c, pl.ds(r0, RPT)])


@functools.partial(
    pl.kernel, mesh=_mesh,
    out_type=jax.ShapeDtypeStruct((NC, R, D), jnp.float32),
    scratch_types=[
        pltpu.VMEM((2, 2, CH), jnp.int32),
        pltpu.VMEM((2, CH, D), jnp.float32),
        pltpu.VMEM_SHARED((R, D), jnp.float32),
        pltpu.SemaphoreType.DMA,
        pltpu.SemaphoreType.DMA,
    ],
)
def _sc_seg(idx_hbm, h_hbm, zh_hbm, out_h,
            idxv, rowsv, acc_h, sem0, sem1):
    c = lax.axis_index("c")
    s = lax.axis_index("s")
    w = c * NS + s
    r0 = s * RPT
    pltpu.sync_copy(zh_hbm.at[pl.ds(r0, RPT)], acc_h.at[pl.ds(r0, RPT)])
    plsc.subcore_barrier()

    sems = (sem0, sem1)

    def run(nck):
        pltpu.sync_copy(idx_hbm.at[w, 0], idxv.at[0])
        pltpu.async_copy(h_hbm.at[idxv.at[0, 0]], rowsv.at[0], sem0)

        def body(k, carry):
            for b in range(2):
                nb = 1 - b
                j = k * 2 + b
                pltpu.sync_copy(idx_hbm.at[w, j + 1], idxv.at[nb])
                pltpu.async_copy(h_hbm.at[idxv.at[nb, 0]], rowsv.at[nb],
                                 sems[nb])
                pltpu.make_async_copy(
                    h_hbm.at[idxv.at[b, 0]], rowsv.at[b], sems[b]).wait()
                pltpu.sync_copy(rowsv.at[b], acc_h.at[idxv.at[b, 1]],
                                add=True)
            return carry

        lax.fori_loop(0, (nck - 1) // 2, body, 0)
        pltpu.make_async_copy(h_hbm.at[idxv.at[0, 0]], rowsv.at[0],
                              sem0).wait()
        pltpu.sync_copy(rowsv.at[0], acc_h.at[idxv.at[0, 1]], add=True)

    @pl.when(c == 0)
    def _():
        run(NCKA)

    @pl.when(c == 1)
    def _():
        run(NCKB)

    plsc.subcore_barrier()
    pltpu.sync_copy(acc_h.at[pl.ds(r0, RPT)], out_h.at[c, pl.ds(r0, RPT)])


@functools.partial(
    pl.kernel, mesh=_mesh,
    out_type=jax.ShapeDtypeStruct((E_PAD * NCLS,), jnp.float32),
    scratch_types=[
        pltpu.VMEM((NCHUNK, CH), jnp.int32),
        pltpu.VMEM((NCHUNK, CH), jnp.int32),
        pltpu.VMEM((N * NCLS,), jnp.float32),
        pltpu.VMEM((N * NCLS,), jnp.float32),
        pltpu.VMEM((CH * NCLS,), jnp.float32),
    ],
    compiler_params=pltpu.CompilerParams(needs_layout_passes=False),
)
def _sc_head(i0_hbm, i1_hbm, p1_hbm, p2_hbm, out_hbm,
             i0v, i1v, p1v, p2v, outv):
    c = lax.axis_index("c")
    s = lax.axis_index("s")
    w = c * NS + s
    pltpu.sync_copy(i0_hbm.at[w], i0v)
    pltpu.sync_copy(i1_hbm.at[w], i1v)
    pltpu.sync_copy(p1_hbm, p1v)
    pltpu.sync_copy(p2_hbm, p2v)
    lane = lax.iota(jnp.int32, 16)

    def body(j, carry):
        for g in range(CH // 16):
            a = i0v[j, pl.ds(g * 16, 16)] * 2
            b = i1v[j, pl.ds(g * 16, 16)] * 2
            c0 = plsc.load_gather(p1v, [a]) + plsc.load_gather(p2v, [b])
            c1 = plsc.load_gather(p1v, [a + 1]) + plsc.load_gather(p2v, [b + 1])
            pos = lane * 2 + (g * 32)
            plsc.store_scatter(outv, [pos], c0)
            plsc.store_scatter(outv, [pos + 1], c1)
        pltpu.sync_copy(outv, out_hbm.at[pl.ds((w * EPW + j * CH) * NCLS,
                                               CH * NCLS)])
        return carry

    lax.fori_loop(0, NCHUNK, body, 0)


_GRID = 10
_BR = N // _GRID

_row_block = lambda d: pl.BlockSpec((_BR, d), lambda i: (i, 0))
_part_block = lambda d: pl.BlockSpec((NC, _BR, d), lambda i: (0, i, 0))
_full = lambda r, d: pl.BlockSpec((r, d), lambda i: (0, 0))


def _tc_pre_body(x_ref, w_ref, b_ref, o_ref):
    o_ref[...] = jnp.maximum(
        jax.lax.dot(x_ref[...], w_ref[...],
                    preferred_element_type=jnp.float32) + b_ref[...], 0.0)


def _tc_pre(x, w, b):
    return pl.pallas_call(
        _tc_pre_body,
        grid=(_GRID,),
        in_specs=[_row_block(D), _full(D, D), _full(1, D)],
        out_specs=_row_block(D),
        out_shape=jax.ShapeDtypeStruct((N, D), jnp.float32),
    )(x, w, b)


def _combine(h, gh, st, wa, wb, wc, mb):
    g = gh[0] + gh[1]
    efa = st[0, :, :DE] + st[1, :, :DE]
    deg = st[0, :, DE:DE + 1] + st[1, :, DE:DE + 1]
    mm = jax.lax.dot(h, wa, preferred_element_type=jnp.float32) + mb
    return (deg * mm
            + jax.lax.dot(g, wb, preferred_element_type=jnp.float32)
            + jax.lax.dot(efa, wc, preferred_element_type=jnp.float32))


def _tc_mid_body(h_ref, gh_ref, st_ref, wa_ref, wb_ref, wc_ref,
                 mb_ref, w2_ref, b2_ref, o_ref):
    x1 = jnp.maximum(
        _combine(h_ref[...], gh_ref[...], st_ref[...],
                 wa_ref[...], wb_ref[...], wc_ref[...], mb_ref[...]), 0.0)
    o_ref[...] = jnp.maximum(
        jax.lax.dot(x1, w2_ref[...], preferred_element_type=jnp.float32)
        + b2_ref[...], 0.0)


def _tc_mid(h, gh, st, wa, wb, wc, mb, w2, b2):
    return pl.pallas_call(
        _tc_mid_body,
        grid=(_GRID,),
        in_specs=[_row_block(D), _part_block(D), _part_block(D),
                  _full(D, D), _full(D, D), _full(DE, D),
                  _full(1, D), _full(D, D), _full(1, D)],
        out_specs=_row_block(D),
        out_shape=jax.ShapeDtypeStruct((N, D), jnp.float32),
    )(h, gh, st, wa, wb, wc, mb, w2, b2)


def _tc_fin_body(h_ref, gh_ref, st_ref, wa_ref, wb_ref, wc_ref,
                 mb_ref, la_ref, lb_ref, lbias_ref, p1_ref, p2_ref):
    x2 = _combine(h_ref[...], gh_ref[...], st_ref[...],
                  wa_ref[...], wb_ref[...], wc_ref[...], mb_ref[...])
    p1_ref[...] = (jax.lax.dot(x2, la_ref[...],
                               preferred_element_type=jnp.float32)
                   + lbias_ref[...])
    p2_ref[...] = jax.lax.dot(x2, lb_ref[...],
                              preferred_element_type=jnp.float32)


def _tc_fin(h, gh, st, wa, wb, wc, mb, la, lb, lbias):
    return pl.pallas_call(
        _tc_fin_body,
        grid=(_GRID,),
        in_specs=[_row_block(D), _part_block(D), _part_block(D),
                  _full(D, D), _full(D, D), _full(DE, D),
                  _full(1, D), _full(D, NCLS), _full(D, NCLS),
                  _full(1, NCLS)],
        out_specs=[_row_block(NCLS), _row_block(NCLS)],
        out_shape=[jax.ShapeDtypeStruct((N, NCLS), jnp.float32),
                   jax.ShapeDtypeStruct((N, NCLS), jnp.float32)],
    )(h, gh, st, wa, wb, wc, mb, la, lb, lbias)


def _pad_idx(x, fill):
    pad = E_PAD - E
    return jnp.concatenate(
        [x.astype(jnp.int32), jnp.full((pad,), fill, jnp.int32)]
    ).reshape(NW, NCHUNK, CH)


def _asym_idx(flat):
    split = NS * NCKA * CH
    a = flat[:split].reshape(NS, NCKA, CH)
    a = jnp.concatenate(
        [a, jnp.zeros((NS, NCKB - NCKA, CH), jnp.int32)], axis=1)
    b = flat[split:].reshape(NS, NCKB, CH)
    return jnp.concatenate([a, b], axis=0)


def kernel(node_feature, edge_index, edge_feature, edge_label_index,
           pre_W1, pre_b1, msg_W1, msg_b1,
           pre_W2, pre_b2, msg_W2, msg_b2,
           lp_W, lp_b):
    dummy = N + (jnp.arange(E_PAD - E, dtype=jnp.int32) % (R - N))
    src_f = jnp.concatenate(
        [edge_index[0].astype(jnp.int32),
         jnp.zeros((E_PAD - E,), jnp.int32)])
    dst_f = jnp.concatenate([edge_index[1].astype(jnp.int32), dummy])
    idx_p = jnp.stack([src_f.reshape(NW, NCHUNK, CH),
                       dst_f.reshape(NW, NCHUNK, CH)],
                      axis=2)
    idx_q = jnp.stack([_asym_idx(src_f), _asym_idx(dst_f)],
                      axis=2)
    i0_p = _pad_idx(edge_label_index[0], 0)
    i1_p = _pad_idx(edge_label_index[1], 0)
    aug = jnp.concatenate(
        [edge_feature, jnp.ones((E, 1), jnp.float32),
         jnp.zeros((E, D - DE - 1), jnp.float32)], axis=1)
    aug_p = jnp.concatenate([aug, jnp.zeros((E_PAD - E, D), jnp.float32)])
    zeros_h = jnp.zeros((R, D), jnp.float32)

    wa1, wb1, wc1 = msg_W1[:D], msg_W1[D:2 * D], msg_W1[2 * D:]
    wa2, wb2, wc2 = msg_W2[:D], msg_W2[D:2 * D], msg_W2[2 * D:]

    st = _sc_stats(idx_p, aug_p, zeros_h)
    h1 = _tc_pre(node_feature, pre_W1, pre_b1.reshape(1, D))
    gh1 = _sc_seg(idx_q, h1, zeros_h)
    h2 = _tc_mid(h1, gh1[:, :N], st[:, :N],
                 wa1, wb1, wc1, msg_b1.reshape(1, D),
                 pre_W2, pre_b2.reshape(1, D))
    gh2 = _sc_seg(idx_q, h2, zeros_h)
    p1, p2 = _tc_fin(h2, gh2[:, :N], st[:, :N],
                     wa2, wb2, wc2, msg_b2.reshape(1, D),
                     lp_W[:D], lp_W[D:], lp_b.reshape(1, NCLS))
    pred_flat = _sc_head(i0_p, i1_p, p1.reshape(-1), p2.reshape(-1))
    return pred_flat[:E * NCLS].reshape(E, NCLS)


# speedup vs baseline: 1.0651x; 1.0651x over previous
import functools

import jax
import jax.numpy as jnp
from jax import lax
from jax.experimental import pallas as pl
from jax.experimental.pallas import tpu as pltpu
from jax.experimental.pallas import tpu_sc as plsc

N = 10000
E = 320000
D = 128
DE = 16
NCLS = 2

NC = 2
NS = 16
NW = NC * NS
CH = 128
NCHUNK = 79
EPW = NCHUNK * CH
E_PAD = NW * EPW
NCKA = 105
NCKB = 53
R = 10112
RPT = R // NS

_mesh = plsc.VectorSubcoreMesh(core_axis_name="c", subcore_axis_name="s")


@functools.partial(
    pl.kernel, mesh=_mesh,
    out_type=jax.ShapeDtypeStruct((NC, R, D), jnp.float32),
    scratch_types=[
        pltpu.VMEM((2, 2, CH), jnp.int32),
        pltpu.VMEM((2, CH, D), jnp.float32),
        pltpu.VMEM_SHARED((R, D), jnp.float32),
        pltpu.SemaphoreType.DMA,
        pltpu.SemaphoreType.DMA,
    ],
)
def _sc_stats(idx_hbm, aug_hbm, zh_hbm, out_st, idxv, rowsv, acc, sem0, sem1):
    c = lax.axis_index("c")
    s = lax.axis_index("s")
    w = c * NS + s
    r0 = s * RPT
    pltpu.sync_copy(zh_hbm.at[pl.ds(r0, RPT)], acc.at[pl.ds(r0, RPT)])
    plsc.subcore_barrier()

    ebase = w * EPW
    sems = (sem0, sem1)
    pltpu.sync_copy(idx_hbm.at[w, 0], idxv.at[0])
    pltpu.async_copy(aug_hbm.at[pl.ds(ebase, CH)], rowsv.at[0], sem0)

    def body(k, carry):
        for b in range(2):
            nb = 1 - b
            j = k * 2 + b
            pltpu.async_copy(aug_hbm.at[pl.ds(ebase + (j + 1) * CH, CH)],
                             rowsv.at[nb], sems[nb])
            pltpu.sync_copy(idx_hbm.at[w, j + 1], idxv.at[nb])
            pltpu.make_async_copy(
                aug_hbm.at[pl.ds(ebase, CH)], rowsv.at[b], sems[b]).wait()
            pltpu.sync_copy(rowsv.at[b], acc.at[idxv.at[b, 1]], add=True)
        return carry

    lax.fori_loop(0, (NCHUNK - 1) // 2, body, 0)
    pltpu.make_async_copy(aug_hbm.at[pl.ds(ebase, CH)], rowsv.at[0],
                          sem0).wait()
    pltpu.sync_copy(rowsv.at[0], acc.at[idxv.at[0, 1]], add=True)
    plsc.subcore_barrier()
    pltpu.sync_copy(acc.at[pl.ds(r0, RPT)], out_st.at[c, pl.ds(r0, RPT)])


@functools.partial(
    pl.kernel, mesh=_mesh,
    out_type=jax.ShapeDtypeStruct((NC, R, D), jnp.float32),
    scratch_types=[
        pltpu.VMEM((2, 2, CH), jnp.int32),
        pltpu.VMEM((2, CH, D), jnp.float32),
        pltpu.VMEM_SHARED((R, D), jnp.float32),
        pltpu.SemaphoreType.DMA,
        pltpu.SemaphoreType.DMA,
    ],
)
def _sc_seg(idx_hbm, h_hbm, zh_hbm, out_h,
            idxv, rowsv, acc_h, sem0, sem1):
    c = lax.axis_index("c")
    s = lax.axis_index("s")
    w = c * NS + s
    r0 = s * RPT
    pltpu.sync_copy(zh_hbm.at[pl.ds(r0, RPT)], acc_h.at[pl.ds(r0, RPT)])
    plsc.subcore_barrier()

    sems = (sem0, sem1)

    def run(nck):
        pltpu.sync_copy(idx_hbm.at[w, 0], idxv.at[0])
        pltpu.async_copy(h_hbm.at[idxv.at[0, 0]], rowsv.at[0], sem0)

        def body(k, carry):
            for b in range(2):
                nb = 1 - b
                j = k * 2 + b
                pltpu.sync_copy(idx_hbm.at[w, j + 1], idxv.at[nb])
                pltpu.async_copy(h_hbm.at[idxv.at[nb, 0]], rowsv.at[nb],
                                 sems[nb])
                pltpu.make_async_copy(
                    h_hbm.at[idxv.at[b, 0]], rowsv.at[b], sems[b]).wait()
                pltpu.sync_copy(rowsv.at[b], acc_h.at[idxv.at[b, 1]],
                                add=True)
            return carry

        lax.fori_loop(0, (nck - 1) // 2, body, 0)
        pltpu.make_async_copy(h_hbm.at[idxv.at[0, 0]], rowsv.at[0],
                              sem0).wait()
        pltpu.sync_copy(rowsv.at[0], acc_h.at[idxv.at[0, 1]], add=True)

    @pl.when(c == 0)
    def _():
        run(NCKA)

    @pl.when(c == 1)
    def _():
        run(NCKB)

    plsc.subcore_barrier()
    pltpu.sync_copy(acc_h.at[pl.ds(r0, RPT)], out_h.at[c, pl.ds(r0, RPT)])


@functools.partial(
    pl.kernel, mesh=_mesh,
    out_type=jax.ShapeDtypeStruct((E_PAD * NCLS,), jnp.float32),
    scratch_types=[
        pltpu.VMEM((NCHUNK, CH), jnp.int32),
        pltpu.VMEM((NCHUNK, CH), jnp.int32),
        pltpu.VMEM((N * NCLS,), jnp.float32),
        pltpu.VMEM((N * NCLS,), jnp.float32),
        pltpu.VMEM((CH * NCLS,), jnp.float32),
    ],
    compiler_params=pltpu.CompilerParams(needs_layout_passes=False),
)
def _sc_head(i0_hbm, i1_hbm, p1_hbm, p2_hbm, out_hbm,
             i0v, i1v, p1v, p2v, outv):
    c = lax.axis_index("c")
    s = lax.axis_index("s")
    w = c * NS + s
    pltpu.sync_copy(i0_hbm.at[w], i0v)
    pltpu.sync_copy(i1_hbm.at[w], i1v)
    pltpu.sync_copy(p1_hbm, p1v)
    pltpu.sync_copy(p2_hbm, p2v)
    lane = lax.iota(jnp.int32, 16)

    def body(j, carry):
        for g in range(CH // 16):
            a = i0v[j, pl.ds(g * 16, 16)] * 2
            b = i1v[j, pl.ds(g * 16, 16)] * 2
            c0 = plsc.load_gather(p1v, [a]) + plsc.load_gather(p2v, [b])
            c1 = plsc.load_gather(p1v, [a + 1]) + plsc.load_gather(p2v, [b + 1])
            pos = lane * 2 + (g * 32)
            plsc.store_scatter(outv, [pos], c0)
            plsc.store_scatter(outv, [pos + 1], c1)
        pltpu.sync_copy(outv, out_hbm.at[pl.ds((w * EPW + j * CH) * NCLS,
                                               CH * NCLS)])
        return carry

    lax.fori_loop(0, NCHUNK, body, 0)


_GRID = 10
_BR = N // _GRID

_row_block = lambda d: pl.BlockSpec((_BR, d), lambda i: (i, 0))
_part_block = lambda d: pl.BlockSpec((NC, _BR, d), lambda i: (0, i, 0))
_full = lambda r, d: pl.BlockSpec((r, d), lambda i: (0, 0))


def _tc_pre_body(x_ref, w_ref, b_ref, o_ref):
    o_ref[...] = jnp.maximum(
        jax.lax.dot(x_ref[...], w_ref[...],
                    preferred_element_type=jnp.float32) + b_ref[...], 0.0)


def _tc_pre(x, w, b):
    return pl.pallas_call(
        _tc_pre_body,
        grid=(_GRID,),
        in_specs=[_row_block(D), _full(D, D), _full(1, D)],
        out_specs=_row_block(D),
        out_shape=jax.ShapeDtypeStruct((N, D), jnp.float32),
    )(x, w, b)


def _combine(h, gh, st, wa, wb, wc, mb):
    g = gh[0] + gh[1]
    efa = st[0, :, :DE] + st[1, :, :DE]
    deg = st[0, :, DE:DE + 1] + st[1, :, DE:DE + 1]
    mm = jax.lax.dot(h, wa, preferred_element_type=jnp.float32) + mb
    return (deg * mm
            + jax.lax.dot(g, wb, preferred_element_type=jnp.float32)
            + jax.lax.dot(efa, wc, preferred_element_type=jnp.float32))


def _tc_mid_body(h_ref, gh_ref, st_ref, wa_ref, wb_ref, wc_ref,
                 mb_ref, w2_ref, b2_ref, o_ref):
    x1 = jnp.maximum(
        _combine(h_ref[...], gh_ref[...], st_ref[...],
                 wa_ref[...], wb_ref[...], wc_ref[...], mb_ref[...]), 0.0)
    o_ref[...] = jnp.maximum(
        jax.lax.dot(x1, w2_ref[...], preferred_element_type=jnp.float32)
        + b2_ref[...], 0.0)


def _tc_mid(h, gh, st, wa, wb, wc, mb, w2, b2):
    return pl.pallas_call(
        _tc_mid_body,
        grid=(_GRID,),
        in_specs=[_row_block(D), _part_block(D), _part_block(D),
                  _full(D, D), _full(D, D), _full(DE, D),
                  _full(1, D), _full(D, D), _full(1, D)],
        out_specs=_row_block(D),
        out_shape=jax.ShapeDtypeStruct((N, D), jnp.float32),
    )(h, gh, st, wa, wb, wc, mb, w2, b2)


def _tc_fin_body(h_ref, gh_ref, st_ref, wa_ref, wb_ref, wc_ref,
                 mb_ref, la_ref, lb_ref, lbias_ref, p1_ref, p2_ref):
    x2 = _combine(h_ref[...], gh_ref[...], st_ref[...],
                  wa_ref[...], wb_ref[...], wc_ref[...], mb_ref[...])
    p1_ref[...] = (jax.lax.dot(x2, la_ref[...],
                               preferred_element_type=jnp.float32)
                   + lbias_ref[...])
    p2_ref[...] = jax.lax.dot(x2, lb_ref[...],
                              preferred_element_type=jnp.float32)


def _tc_fin(h, gh, st, wa, wb, wc, mb, la, lb, lbias):
    return pl.pallas_call(
        _tc_fin_body,
        grid=(_GRID,),
        in_specs=[_row_block(D), _part_block(D), _part_block(D),
                  _full(D, D), _full(D, D), _full(DE, D),
                  _full(1, D), _full(D, NCLS), _full(D, NCLS),
                  _full(1, NCLS)],
        out_specs=[_row_block(NCLS), _row_block(NCLS)],
        out_shape=[jax.ShapeDtypeStruct((N, NCLS), jnp.float32),
                   jax.ShapeDtypeStruct((N, NCLS), jnp.float32)],
    )(h, gh, st, wa, wb, wc, mb, la, lb, lbias)


def _pad_idx(x, fill):
    pad = E_PAD - E
    return jnp.concatenate(
        [x.astype(jnp.int32), jnp.full((pad,), fill, jnp.int32)]
    ).reshape(NW, NCHUNK, CH)


def _asym_idx(flat):
    m = max(NCKA, NCKB)
    split = NS * NCKA * CH
    a = flat[:split].reshape(NS, NCKA, CH)
    b = flat[split:].reshape(NS, NCKB, CH)
    pad = lambda x, n: jnp.concatenate(
        [x, jnp.zeros((NS, n, CH), jnp.int32)], axis=1) if n else x
    return jnp.concatenate([pad(a, m - NCKA), pad(b, m - NCKB)], axis=0)


def kernel(node_feature, edge_index, edge_feature, edge_label_index,
           pre_W1, pre_b1, msg_W1, msg_b1,
           pre_W2, pre_b2, msg_W2, msg_b2,
           lp_W, lp_b):
    dummy = N + (jnp.arange(E_PAD - E, dtype=jnp.int32) % (R - N))
    src_f = jnp.concatenate(
        [edge_index[0].astype(jnp.int32),
         jnp.zeros((E_PAD - E,), jnp.int32)])
    dst_f = jnp.concatenate([edge_index[1].astype(jnp.int32), dummy])
    idx_p = jnp.stack([src_f.reshape(NW, NCHUNK, CH),
                       dst_f.reshape(NW, NCHUNK, CH)],
                      axis=2)
    idx_q = jnp.stack([_asym_idx(src_f), _asym_idx(dst_f)],
                      axis=2)
    i0_p = _pad_idx(edge_label_index[0], 0)
    i1_p = _pad_idx(edge_label_index[1], 0)
    aug = jnp.concatenate(
        [edge_feature, jnp.ones((E, 1), jnp.float32),
         jnp.zeros((E, D - DE - 1), jnp.float32)], axis=1)
    aug_p = jnp.concatenate([aug, jnp.zeros((E_PAD - E, D), jnp.float32)])
    zeros_h = jnp.zeros((R, D), jnp.float32)

    wa1, wb1, wc1 = msg_W1[:D], msg_W1[D:2 * D], msg_W1[2 * D:]
    wa2, wb2, wc2 = msg_W2[:D], msg_W2[D:2 * D], msg_W2[2 * D:]

    st = _sc_stats(idx_p, aug_p, zeros_h)
    h1 = _tc_pre(node_feature, pre_W1, pre_b1.reshape(1, D))
    gh1 = _sc_seg(idx_q, h1, zeros_h)
    h2 = _tc_mid(h1, gh1[:, :N], st[:, :N],
                 wa1, wb1, wc1, msg_b1.reshape(1, D),
                 pre_W2, pre_b2.reshape(1, D))
    gh2 = _sc_seg(idx_q, h2, zeros_h)
    p1, p2 = _tc_fin(h2, gh2[:, :N], st[:, :N],
                     wa2, wb2, wc2, msg_b2.reshape(1, D),
                     lp_W[:D], lp_W[D:], lp_b.reshape(1, NCLS))
    pred_flat = _sc_head(i0_p, i1_p, p1.reshape(-1), p2.reshape(-1))
    return pred_flat[:E * NCLS].reshape(E, NCLS)
